# Initial kernel scaffold; baseline (speedup 1.0000x reference)
import jax
import jax.numpy as jnp
from jax.experimental import pallas as pl


def kernel(x, emb):
    raise NotImplementedError("write your pallas kernel here")


# speedup vs baseline: 1.3271x; 1.3271x over previous
import functools

import jax
import jax.numpy as jnp
from jax import lax
from jax.experimental import pallas as pl
from jax.experimental.pallas import tpu as pltpu
from jax.experimental.pallas import tpu_sc as plsc

NUM_CODES = 8192
CODE_DIM = 32
COMMITMENT_COST = 0.25


_BLK = 256


def _argmin_body(x_ref, et_ref, codes_ref, loss_ref):
    i = pl.program_id(0)
    x_blk = x_ref[...]
    et = et_ref[...]
    e2 = jnp.sum(et * et, axis=0, keepdims=True)
    x2 = jnp.sum(x_blk * x_blk, axis=1, keepdims=True)
    dots = jnp.dot(x_blk.astype(jnp.bfloat16), et.astype(jnp.bfloat16),
                   preferred_element_type=jnp.float32)
    dist = (x2 + e2) - 2.0 * dots
    half = NUM_CODES // 2
    d_lo, d_hi = dist[:, :half], dist[:, half:]
    idx_half = lax.broadcasted_iota(jnp.int32, d_lo.shape, 1)
    v_lo = jnp.min(d_lo, axis=1, keepdims=True)
    w_lo = jnp.min(jnp.where(d_lo <= v_lo, idx_half, NUM_CODES), axis=1)
    v_hi = jnp.min(d_hi, axis=1, keepdims=True)
    w_hi = jnp.min(jnp.where(d_hi <= v_hi, idx_half, NUM_CODES), axis=1) + half
    v_lo = v_lo[:, 0]
    v_hi = v_hi[:, 0]
    take_hi = v_hi < v_lo.astype(jnp.bfloat16).astype(jnp.float32)
    codes = jnp.where(take_hi, w_hi, w_lo)
    codes_ref[0, 0, :] = codes
    part = jnp.sum(jnp.where(take_hi, v_hi, v_lo))

    @pl.when(i == 0)
    def _():
        loss_ref[0, 0] = 0.0

    loss_ref[0, 0] += part


def _tc_argmin(flat_x, emb_t):
    n = flat_x.shape[0]
    grid = n // _BLK
    codes3d, loss = pl.pallas_call(
        _argmin_body,
        grid=(grid,),
        in_specs=[
            pl.BlockSpec((_BLK, CODE_DIM), lambda i: (i, 0)),
            pl.BlockSpec((CODE_DIM, NUM_CODES), lambda i: (0, 0)),
        ],
        out_specs=[
            pl.BlockSpec((1, 1, _BLK), lambda i: (i, 0, 0)),
            pl.BlockSpec((1, 1), lambda i: (0, 0),
                         memory_space=pltpu.SMEM),
        ],
        out_shape=[
            jax.ShapeDtypeStruct((grid, 1, _BLK), jnp.int32),
            jax.ShapeDtypeStruct((1, 1), jnp.float32),
        ],
    )(flat_x, emb_t)
    return codes3d.reshape(-1), loss[0, 0]


_IDX_CHUNK = 128


def _sc_gather_body(table_hbm, idx_hbm, out_hbm, idx_v, rows_v, sem,
                    *, n_chunk, num_cores):
    wid = lax.axis_index("s") * num_cores + lax.axis_index("c")
    pltpu.sync_copy(idx_hbm.at[wid], idx_v)
    copies = []
    for j in range(n_chunk):
        copies.append(
            pltpu.async_copy(
                table_hbm.at[idx_v.at[j]],
                rows_v.at[pl.ds(j * _IDX_CHUNK, _IDX_CHUNK)],
                sem,
            ))
    for c in copies:
        c.wait()
    pltpu.sync_copy(rows_v, out_hbm.at[wid])


def _sc_gather(emb, codes):
    info = plsc.get_sparse_core_info()
    nw = info.num_cores * info.num_subcores
    n = codes.shape[0]
    b_per_w = n // nw
    n_chunk = b_per_w // _IDX_CHUNK
    mesh = plsc.VectorSubcoreMesh(core_axis_name="c", subcore_axis_name="s")
    kfn = pl.kernel(
        functools.partial(_sc_gather_body, n_chunk=n_chunk,
                          num_cores=info.num_cores),
        mesh=mesh,
        out_type=jax.ShapeDtypeStruct((nw, b_per_w, CODE_DIM), jnp.float32),
        scratch_types=[
            pltpu.VMEM((n_chunk, _IDX_CHUNK), jnp.int32),
            pltpu.VMEM((b_per_w, CODE_DIM), jnp.float32),
            pltpu.SemaphoreType.DMA,
        ],
        compiler_params=pltpu.CompilerParams(use_tc_tiling_on_sc=False),
    )
    out = kfn(emb, codes.reshape(nw, n_chunk, _IDX_CHUNK))
    return out.reshape(n, CODE_DIM)


def kernel(x, emb):
    B, D, H, W = x.shape
    flat_x = jnp.transpose(x, (0, 2, 3, 1)).reshape(-1, D)
    codes, loss_sum = _tc_argmin(flat_x, emb.T)
    flat_x_q = _sc_gather(emb, codes)
    x_q = jnp.transpose(flat_x_q.reshape(B, H, W, D), (0, 3, 1, 2))
    vq_loss = loss_sum * ((1.0 + COMMITMENT_COST) / (B * D * H * W))
    x_q_st = x + lax.stop_gradient(x_q - x)
    codes_map = codes.reshape(B, H, W)
    return (x_q_st, vq_loss, codes_map)


# speedup vs baseline: 1.3458x; 1.0141x over previous
import functools

import jax
import jax.numpy as jnp
from jax import lax
from jax.experimental import pallas as pl
from jax.experimental.pallas import tpu as pltpu
from jax.experimental.pallas import tpu_sc as plsc

NUM_CODES = 8192
CODE_DIM = 32
COMMITMENT_COST = 0.25


_BLK = 512


def _argmin_body(x_ref, et_ref, codes_ref, loss_ref):
    i = pl.program_id(0)
    x_blk = x_ref[...]
    et = et_ref[...]
    e2 = jnp.sum(et * et, axis=0, keepdims=True)
    x2 = jnp.sum(x_blk * x_blk, axis=1, keepdims=True)
    dots = jnp.dot(x_blk.astype(jnp.bfloat16), et.astype(jnp.bfloat16),
                   preferred_element_type=jnp.float32)
    dist = (x2 + e2) - 2.0 * dots
    half = NUM_CODES // 2
    d_lo, d_hi = dist[:, :half], dist[:, half:]
    idx_half = lax.broadcasted_iota(jnp.int32, d_lo.shape, 1)
    v_lo = jnp.min(d_lo, axis=1, keepdims=True)
    w_lo = jnp.min(jnp.where(d_lo <= v_lo, idx_half, NUM_CODES), axis=1)
    v_hi = jnp.min(d_hi, axis=1, keepdims=True)
    w_hi = jnp.min(jnp.where(d_hi <= v_hi, idx_half, NUM_CODES), axis=1) + half
    v_lo = v_lo[:, 0]
    v_hi = v_hi[:, 0]
    take_hi = v_hi < v_lo.astype(jnp.bfloat16).astype(jnp.float32)
    codes = jnp.where(take_hi, w_hi, w_lo)
    codes_ref[0, 0, :] = codes
    part = jnp.sum(jnp.where(take_hi, v_hi, v_lo))

    @pl.when(i == 0)
    def _():
        loss_ref[0, 0] = 0.0

    loss_ref[0, 0] += part


def _tc_argmin(flat_x, emb_t):
    n = flat_x.shape[0]
    grid = n // _BLK
    codes3d, loss = pl.pallas_call(
        _argmin_body,
        grid=(grid,),
        in_specs=[
            pl.BlockSpec((_BLK, CODE_DIM), lambda i: (i, 0)),
            pl.BlockSpec((CODE_DIM, NUM_CODES), lambda i: (0, 0)),
        ],
        out_specs=[
            pl.BlockSpec((1, 1, _BLK), lambda i: (i, 0, 0)),
            pl.BlockSpec((1, 1), lambda i: (0, 0),
                         memory_space=pltpu.SMEM),
        ],
        out_shape=[
            jax.ShapeDtypeStruct((grid, 1, _BLK), jnp.int32),
            jax.ShapeDtypeStruct((1, 1), jnp.float32),
        ],
    )(flat_x, emb_t)
    return codes3d.reshape(-1), loss[0, 0]


_IDX_CHUNK = 128


def _sc_gather_body(table_hbm, idx_hbm, out_hbm, idx_v, rows_v, sem,
                    *, n_chunk, num_cores):
    wid = lax.axis_index("s") * num_cores + lax.axis_index("c")
    pltpu.sync_copy(idx_hbm.at[wid], idx_v)
    copies = []
    for j in range(n_chunk):
        copies.append(
            pltpu.async_copy(
                table_hbm.at[idx_v.at[j]],
                rows_v.at[pl.ds(j * _IDX_CHUNK, _IDX_CHUNK)],
                sem,
            ))
    for c in copies:
        c.wait()
    pltpu.sync_copy(rows_v, out_hbm.at[wid])


def _sc_gather(emb, codes):
    info = plsc.get_sparse_core_info()
    nw = info.num_cores * info.num_subcores
    n = codes.shape[0]
    b_per_w = n // nw
    n_chunk = b_per_w // _IDX_CHUNK
    mesh = plsc.VectorSubcoreMesh(core_axis_name="c", subcore_axis_name="s")
    kfn = pl.kernel(
        functools.partial(_sc_gather_body, n_chunk=n_chunk,
                          num_cores=info.num_cores),
        mesh=mesh,
        out_type=jax.ShapeDtypeStruct((nw, b_per_w, CODE_DIM), jnp.float32),
        scratch_types=[
            pltpu.VMEM((n_chunk, _IDX_CHUNK), jnp.int32),
            pltpu.VMEM((b_per_w, CODE_DIM), jnp.float32),
            pltpu.SemaphoreType.DMA,
        ],
        compiler_params=pltpu.CompilerParams(use_tc_tiling_on_sc=False),
    )
    out = kfn(emb, codes.reshape(nw, n_chunk, _IDX_CHUNK))
    return out.reshape(n, CODE_DIM)


def kernel(x, emb):
    B, D, H, W = x.shape
    flat_x = jnp.transpose(x, (0, 2, 3, 1)).reshape(-1, D)
    codes, loss_sum = _tc_argmin(flat_x, emb.T)
    flat_x_q = _sc_gather(emb, codes)
    x_q = jnp.transpose(flat_x_q.reshape(B, H, W, D), (0, 3, 1, 2))
    vq_loss = loss_sum * ((1.0 + COMMITMENT_COST) / (B * D * H * W))
    x_q_st = x + lax.stop_gradient(x_q - x)
    codes_map = codes.reshape(B, H, W)
    return (x_q_st, vq_loss, codes_map)
